# Initial kernel scaffold; baseline (speedup 1.0000x reference)
import jax
import jax.numpy as jnp
from jax.experimental import pallas as pl


def kernel(x, W, gamma, beta):
    raise NotImplementedError("write your pallas kernel here")


# speedup vs baseline: 3.6640x; 3.6640x over previous
import functools

import jax
import jax.numpy as jnp
from jax import lax
from jax.experimental import pallas as pl
from jax.experimental.pallas import tpu as pltpu
from jax.experimental.pallas import tpu_sc as plsc

VOCAB = 1000
DIM = 128
BATCH = 16384
FIELDS = 26
LN_EPS = 1e-5

ROWS = BATCH * FIELDS
NW = 32
RPW = ROWS // NW
CHUNK = 128
NCH = RPW // CHUNK


def _ln_table_kernel(w_ref, g_ref, b_ref, o_ref):
    w = w_ref[...]
    mean = jnp.mean(w, axis=1, keepdims=True)
    d = w - mean
    var = jnp.mean(d * d, axis=1, keepdims=True)
    o_ref[...] = d * lax.rsqrt(var + LN_EPS) * g_ref[...] + b_ref[...]


def _normalize_table(W, gamma, beta):
    return pl.pallas_call(
        _ln_table_kernel,
        out_shape=jax.ShapeDtypeStruct((VOCAB, DIM), jnp.float32),
    )(W, gamma.reshape(1, DIM), beta.reshape(1, DIM))


def _sc_gather_body(nt_hbm, idx_hbm, out_hbm, idx_v, buf_v, g0, g1, s0, s1):
    nc = 2
    wid = lax.axis_index("s") * nc + lax.axis_index("c")
    row_base = wid * RPW
    gsem = (g0, g1)
    ssem = (s0, s1)

    pltpu.sync_copy(idx_hbm.at[wid], idx_v)

    def gather(c, b):
        return pltpu.make_async_copy(
            nt_hbm.at[idx_v.at[c]], buf_v.at[b], gsem[b])

    def scatter(c, b):
        return pltpu.make_async_copy(
            buf_v.at[b], out_hbm.at[pl.ds(row_base + c * CHUNK, CHUNK)],
            ssem[b])

    gather(0, 0).start()
    gather(0, 0).wait()
    gather(1, 1).start()
    scatter(0, 0).start()
    gather(1, 1).wait()
    scatter(0, 0).wait()
    gather(2, 0).start()
    scatter(1, 1).start()

    def pair(g, _):
        c0 = 2 * g
        c1 = c0 + 1
        gather(c0, 0).wait()
        scatter(c1 - 2, 1).wait()
        gather(c1, 1).start()
        scatter(c0, 0).start()
        gather(c1, 1).wait()
        scatter(c0, 0).wait()
        gather(c1 + 1, 0).start()
        scatter(c1, 1).start()
        return 0

    lax.fori_loop(1, NCH // 2 - 1, pair, 0)

    gather(NCH - 2, 0).wait()
    scatter(NCH - 3, 1).wait()
    gather(NCH - 1, 1).start()
    scatter(NCH - 2, 0).start()
    gather(NCH - 1, 1).wait()
    scatter(NCH - 2, 0).wait()
    scatter(NCH - 1, 1).start()
    scatter(NCH - 1, 1).wait()


@functools.partial(
    pl.kernel,
    out_type=jax.ShapeDtypeStruct((ROWS, DIM), jnp.float32),
    mesh=plsc.VectorSubcoreMesh(core_axis_name="c", subcore_axis_name="s"),
    scratch_types=[
        pltpu.VMEM((NCH, CHUNK), jnp.int32),
        pltpu.VMEM((2, CHUNK, DIM), jnp.float32),
        pltpu.SemaphoreType.DMA,
        pltpu.SemaphoreType.DMA,
        pltpu.SemaphoreType.DMA,
        pltpu.SemaphoreType.DMA,
    ],
)
def _sc_gather(nt_hbm, idx_hbm, out_hbm, idx_v, buf_v, g0, g1, s0, s1):
    _sc_gather_body(nt_hbm, idx_hbm, out_hbm, idx_v, buf_v, g0, g1, s0, s1)


def kernel(x, W, gamma, beta):
    nt = _normalize_table(W, gamma, beta)
    idx = x.astype(jnp.int32).reshape(NW, NCH, CHUNK)
    out = _sc_gather(nt, idx)
    return out.reshape(BATCH, FIELDS, DIM)


# speedup vs baseline: 5.8611x; 1.5997x over previous
import functools

import jax
import jax.numpy as jnp
from jax import lax
from jax.experimental import pallas as pl
from jax.experimental.pallas import tpu as pltpu
from jax.experimental.pallas import tpu_sc as plsc

VOCAB = 1000
DIM = 128
BATCH = 16384
FIELDS = 26
LN_EPS = 1e-5

NW = 32
BPW = BATCH // NW
K = 8
NB = BPW // K


def _ln_table_kernel(w_ref, g_ref, b_ref, o_ref):
    w = w_ref[...]
    mean = jnp.mean(w, axis=1, keepdims=True)
    d = w - mean
    var = jnp.mean(d * d, axis=1, keepdims=True)
    o_ref[...] = d * lax.rsqrt(var + LN_EPS) * g_ref[...] + b_ref[...]


def _normalize_table(W, gamma, beta):
    return pl.pallas_call(
        _ln_table_kernel,
        out_shape=jax.ShapeDtypeStruct((VOCAB, DIM), jnp.float32),
    )(W, gamma.reshape(1, DIM), beta.reshape(1, DIM))


def _sc_gather_body(nt_hbm, x_hbm, out_hbm, idx_v, buf_v, g0, g1, s0, s1):
    nc = 2
    wid = lax.axis_index("s") * nc + lax.axis_index("c")
    row_base = wid * BPW
    gsem = (g0, g1)
    ssem = (s0, s1)

    pltpu.sync_copy(x_hbm.at[pl.ds(row_base, BPW)], idx_v)

    def row_gather(t, b, j):
        return pltpu.make_async_copy(
            nt_hbm.at[idx_v.at[t * K + j]], buf_v.at[b, j], gsem[b])

    def fire(t, b):
        for j in range(K):
            row_gather(t, b, j).start()

    def drain(t, b):
        for j in range(K):
            row_gather(t, b, j).wait()

    def scatter(t, b):
        return pltpu.make_async_copy(
            buf_v.at[b], out_hbm.at[pl.ds(row_base + t * K, K)], ssem[b])

    fire(0, 0)
    drain(0, 0)
    fire(1, 1)
    scatter(0, 0).start()
    drain(1, 1)
    scatter(0, 0).wait()
    fire(2, 0)
    scatter(1, 1).start()

    def pair(g, _):
        t0 = 2 * g
        t1 = t0 + 1
        drain(t0, 0)
        scatter(t1 - 2, 1).wait()
        fire(t1, 1)
        scatter(t0, 0).start()
        drain(t1, 1)
        scatter(t0, 0).wait()
        fire(t1 + 1, 0)
        scatter(t1, 1).start()
        return 0

    lax.fori_loop(1, NB // 2 - 1, pair, 0)

    drain(NB - 2, 0)
    scatter(NB - 3, 1).wait()
    fire(NB - 1, 1)
    scatter(NB - 2, 0).start()
    drain(NB - 1, 1)
    scatter(NB - 2, 0).wait()
    scatter(NB - 1, 1).start()
    scatter(NB - 1, 1).wait()


@functools.partial(
    pl.kernel,
    out_type=jax.ShapeDtypeStruct((BATCH, FIELDS, DIM), jnp.float32),
    mesh=plsc.VectorSubcoreMesh(core_axis_name="c", subcore_axis_name="s"),
    scratch_types=[
        pltpu.VMEM((BPW, FIELDS), jnp.int32),
        pltpu.VMEM((2, K, FIELDS, DIM), jnp.float32),
        pltpu.SemaphoreType.DMA,
        pltpu.SemaphoreType.DMA,
        pltpu.SemaphoreType.DMA,
        pltpu.SemaphoreType.DMA,
    ],
)
def _sc_gather(nt_hbm, x_hbm, out_hbm, idx_v, buf_v, g0, g1, s0, s1):
    _sc_gather_body(nt_hbm, x_hbm, out_hbm, idx_v, buf_v, g0, g1, s0, s1)


def kernel(x, W, gamma, beta):
    nt = _normalize_table(W, gamma, beta)
    return _sc_gather(nt, x.astype(jnp.int32))


# speedup vs baseline: 5.8679x; 1.0012x over previous
import functools

import jax
import jax.numpy as jnp
from jax import lax
from jax.experimental import pallas as pl
from jax.experimental.pallas import tpu as pltpu
from jax.experimental.pallas import tpu_sc as plsc

VOCAB = 1000
DIM = 128
BATCH = 16384
FIELDS = 26
LN_EPS = 1e-5

NW = 32
BPW = BATCH // NW
K = 8
NB = BPW // K


def _ln_table_kernel(w_ref, g_ref, b_ref, o_ref):
    w = w_ref[...]
    mean = jnp.mean(w, axis=1, keepdims=True)
    d = w - mean
    var = jnp.mean(d * d, axis=1, keepdims=True)
    o_ref[...] = d * lax.rsqrt(var + LN_EPS) * g_ref[...] + b_ref[...]


def _normalize_table(W, gamma, beta):
    return pl.pallas_call(
        _ln_table_kernel,
        out_shape=jax.ShapeDtypeStruct((VOCAB, DIM), jnp.float32),
    )(W, gamma.reshape(1, DIM), beta.reshape(1, DIM))


def _sc_gather_body(nt_hbm, x_hbm, out_hbm, idx_v, buf_v, g0, g1, s0, s1):
    nc = 2
    wid = lax.axis_index("s") * nc + lax.axis_index("c")
    row_base = wid * BPW
    gsem = (g0, g1)
    ssem = (s0, s1)

    pltpu.sync_copy(x_hbm.at[pl.ds(row_base, BPW)], idx_v)

    def row_gather(t, b, j):
        return pltpu.make_async_copy(
            nt_hbm.at[idx_v.at[t * K + j]], buf_v.at[b, j], gsem[b])

    def fire(t, b):
        for j in range(K):
            row_gather(t, b, j).start()

    def drain(t, b):
        for j in range(K):
            row_gather(t, b, j).wait()

    def scatter(t, b):
        return pltpu.make_async_copy(
            buf_v.at[b], out_hbm.at[pl.ds(row_base + t * K, K)], ssem[b])

    fire(0, 0)
    drain(0, 0)
    fire(1, 1)
    scatter(0, 0).start()
    drain(1, 1)
    scatter(0, 0).wait()
    fire(2, 0)
    scatter(1, 1).start()

    def pair(g, _):
        t0 = 2 * g
        t1 = t0 + 1
        drain(t0, 0)
        scatter(t1 - 2, 1).wait()
        fire(t1, 1)
        scatter(t0, 0).start()
        drain(t1, 1)
        scatter(t0, 0).wait()
        fire(t1 + 1, 0)
        scatter(t1, 1).start()
        return 0

    lax.fori_loop(1, NB // 2 - 1, pair, 0)

    drain(NB - 2, 0)
    scatter(NB - 3, 1).wait()
    fire(NB - 1, 1)
    scatter(NB - 2, 0).start()
    drain(NB - 1, 1)
    scatter(NB - 2, 0).wait()
    scatter(NB - 1, 1).start()
    scatter(NB - 1, 1).wait()


@functools.partial(
    pl.kernel,
    out_type=jax.ShapeDtypeStruct((BATCH, FIELDS, DIM), jnp.float32),
    mesh=plsc.VectorSubcoreMesh(core_axis_name="c", subcore_axis_name="s"),
    compiler_params=pltpu.CompilerParams(use_tc_tiling_on_sc=True),
    scratch_types=[
        pltpu.VMEM((BPW, FIELDS), jnp.int32),
        pltpu.VMEM((2, K, FIELDS, DIM), jnp.float32),
        pltpu.SemaphoreType.DMA,
        pltpu.SemaphoreType.DMA,
        pltpu.SemaphoreType.DMA,
        pltpu.SemaphoreType.DMA,
    ],
)
def _sc_gather(nt_hbm, x_hbm, out_hbm, idx_v, buf_v, g0, g1, s0, s1):
    _sc_gather_body(nt_hbm, x_hbm, out_hbm, idx_v, buf_v, g0, g1, s0, s1)


def kernel(x, W, gamma, beta):
    nt = _normalize_table(W, gamma, beta)
    return _sc_gather(nt, x.astype(jnp.int32))


# speedup vs baseline: 9.6781x; 1.6493x over previous
import functools

import jax
import jax.numpy as jnp
from jax import lax
from jax.experimental import pallas as pl
from jax.experimental.pallas import tpu as pltpu
from jax.experimental.pallas import tpu_sc as plsc

VOCAB = 1000
DIM = 128
BATCH = 16384
FIELDS = 26
LN_EPS = 1e-5

NW = 32
BPW = BATCH // NW
CK = 128
NR = BPW // CK
NCH = FIELDS * NR


def _ln_table_kernel(w_ref, g_ref, b_ref, o_ref):
    w = w_ref[...]
    mean = jnp.mean(w, axis=1, keepdims=True)
    d = w - mean
    var = jnp.mean(d * d, axis=1, keepdims=True)
    o_ref[...] = d * lax.rsqrt(var + LN_EPS) * g_ref[...] + b_ref[...]


def _normalize_table(W, gamma, beta):
    return pl.pallas_call(
        _ln_table_kernel,
        out_shape=jax.ShapeDtypeStruct((VOCAB, DIM), jnp.float32),
    )(W, gamma.reshape(1, DIM), beta.reshape(1, DIM))


def _sc_gather_body(nt_hbm, xt_hbm, out_hbm, idx_t, buf_v, g0, g1, s0, s1):
    nc = 2
    wid = lax.axis_index("s") * nc + lax.axis_index("c")
    row_base = wid * BPW
    gsem = (g0, g1)
    ssem = (s0, s1)

    pltpu.sync_copy(xt_hbm.at[pl.ds(0, FIELDS), pl.ds(row_base, BPW)], idx_t)

    def gather(c, b):
        f = c // NR
        r = c % NR
        return pltpu.make_async_copy(
            nt_hbm.at[idx_t.at[f, pl.ds(r * CK, CK)]], buf_v.at[b], gsem[b])

    def scatter(c, b):
        f = c // NR
        r = c % NR
        return pltpu.make_async_copy(
            buf_v.at[b],
            out_hbm.at[f, pl.ds(row_base + r * CK, CK)], ssem[b])

    gather(0, 0).start()
    gather(0, 0).wait()
    gather(1, 1).start()
    scatter(0, 0).start()
    gather(1, 1).wait()
    scatter(0, 0).wait()
    gather(2, 0).start()
    scatter(1, 1).start()

    def pair(g, _):
        c0 = 2 * g
        c1 = c0 + 1
        gather(c0, 0).wait()
        scatter(c1 - 2, 1).wait()
        gather(c1, 1).start()
        scatter(c0, 0).start()
        gather(c1, 1).wait()
        scatter(c0, 0).wait()
        gather(c1 + 1, 0).start()
        scatter(c1, 1).start()
        return 0

    lax.fori_loop(1, NCH // 2 - 1, pair, 0)

    gather(NCH - 2, 0).wait()
    scatter(NCH - 3, 1).wait()
    gather(NCH - 1, 1).start()
    scatter(NCH - 2, 0).start()
    gather(NCH - 1, 1).wait()
    scatter(NCH - 2, 0).wait()
    scatter(NCH - 1, 1).start()
    scatter(NCH - 1, 1).wait()


@functools.partial(
    pl.kernel,
    out_type=jax.ShapeDtypeStruct((FIELDS, BATCH, DIM), jnp.float32),
    mesh=plsc.VectorSubcoreMesh(core_axis_name="c", subcore_axis_name="s"),
    scratch_types=[
        pltpu.VMEM((FIELDS, BPW), jnp.int32),
        pltpu.VMEM((2, CK, DIM), jnp.float32),
        pltpu.SemaphoreType.DMA,
        pltpu.SemaphoreType.DMA,
        pltpu.SemaphoreType.DMA,
        pltpu.SemaphoreType.DMA,
    ],
)
def _sc_gather(nt_hbm, xt_hbm, out_hbm, idx_t, buf_v, g0, g1, s0, s1):
    _sc_gather_body(nt_hbm, xt_hbm, out_hbm, idx_t, buf_v, g0, g1, s0, s1)


def kernel(x, W, gamma, beta):
    nt = _normalize_table(W, gamma, beta)
    out_fm = _sc_gather(nt, jnp.transpose(x.astype(jnp.int32)))
    return jnp.transpose(out_fm, (1, 0, 2))
